# baseline (speedup 1.0000x reference)
import functools

import jax
import jax.numpy as jnp
from jax import lax
from jax.experimental import pallas as pl
from jax.experimental.pallas import tpu as pltpu
from jax.experimental.pallas import tpu_sc as plsc

HIDDEN = 2048
NUM_EXPERTS = 8
TOP_K = 2
BLOCK = 1024
LANES = 16


def _gate_block(x_ref, wt_ref, logits_ref):
    logits_ref[...] = jax.lax.dot_general(
        wt_ref[...], x_ref[...], (((1,), (1,)), ((), ())),
        preferred_element_type=jnp.float32)


def _gate_logits_t(x, wt, T):
    return pl.pallas_call(
        _gate_block,
        grid=(T // BLOCK,),
        in_specs=[
            pl.BlockSpec((BLOCK, HIDDEN), lambda i: (i, 0)),
            pl.BlockSpec((NUM_EXPERTS, HIDDEN), lambda i: (0, 0)),
        ],
        out_specs=pl.BlockSpec((NUM_EXPERTS, BLOCK), lambda i: (0, i)),
        out_shape=jax.ShapeDtypeStruct((NUM_EXPERTS, T), jnp.float32),
        compiler_params=pltpu.CompilerParams(
            dimension_semantics=("arbitrary",),
        ),
    )(x, wt)


def _make_sc_topk(T):
    info = plsc.get_sparse_core_info()
    NC, NS = info.num_cores, info.num_subcores
    NW = NC * NS
    chunk = T // NW

    mesh = plsc.VectorSubcoreMesh(core_axis_name="c", subcore_axis_name="s")

    @functools.partial(
        pl.kernel, mesh=mesh,
        out_type=[
            jax.ShapeDtypeStruct((TOP_K, T), jnp.float32),
            jax.ShapeDtypeStruct((TOP_K, T), jnp.int32),
        ],
        scratch_types=[
            pltpu.VMEM((NUM_EXPERTS, chunk), jnp.float32),
            pltpu.VMEM((TOP_K, chunk), jnp.float32),
            pltpu.VMEM((TOP_K, chunk), jnp.int32),
        ],
    )
    def sc_topk(logits_hbm, rw_hbm, idx_hbm, l_v, rw_v, idx_v):
        wid = lax.axis_index("s") * NC + lax.axis_index("c")
        base = wid * chunk
        pltpu.sync_copy(logits_hbm.at[:, pl.ds(base, chunk)], l_v)

        def body(i, _):
            off = i * LANES
            m1 = l_v[0, pl.ds(off, LANES)]
            i1 = jnp.zeros((LANES,), jnp.int32)
            m2 = jnp.full((LANES,), -jnp.inf, jnp.float32)
            i2 = jnp.full((LANES,), NUM_EXPERTS, jnp.int32)
            for e in range(1, NUM_EXPERTS):
                le = l_v[e, pl.ds(off, LANES)]
                gt1 = le > m1
                gt2 = le > m2
                m2 = jnp.where(gt1, m1, jnp.where(gt2, le, m2))
                i2 = jnp.where(gt1, i1, jnp.where(gt2, e, i2))
                m1 = jnp.where(gt1, le, m1)
                i1 = jnp.where(gt1, e, i1)
            e2 = jnp.exp(m2 - m1)
            denom = 1.0 + e2
            rw_v[0, pl.ds(off, LANES)] = 1.0 / denom
            rw_v[1, pl.ds(off, LANES)] = e2 / denom
            idx_v[0, pl.ds(off, LANES)] = i1
            idx_v[1, pl.ds(off, LANES)] = i2
            return 0

        lax.fori_loop(0, chunk // LANES, body, 0)
        pltpu.sync_copy(rw_v, rw_hbm.at[:, pl.ds(base, chunk)])
        pltpu.sync_copy(idx_v, idx_hbm.at[:, pl.ds(base, chunk)])

    return sc_topk


def kernel(hidden_states, W_gate):
    B, S, H = hidden_states.shape
    T = B * S
    x = hidden_states.reshape(T, H)
    wt = W_gate.T

    logits_t = _gate_logits_t(x, wt, T)
    rw_t, idx_t = _make_sc_topk(T)(logits_t)

    return (rw_t.T.reshape(B, S, TOP_K),
            idx_t.T.reshape(B, S, TOP_K),
            logits_t.T.reshape(B, S, NUM_EXPERTS))


# speedup vs baseline: 1.2484x; 1.2484x over previous
import jax
import jax.numpy as jnp
from jax.experimental import pallas as pl
from jax.experimental.pallas import tpu as pltpu

HIDDEN = 2048
NUM_EXPERTS = 8
TOP_K = 2
BLOCK = 1024


def _router_block(x_ref, wt_ref, logits_ref, rw_ref, idx_ref):
    x = x_ref[...]
    wt = wt_ref[...]
    logits_t = jax.lax.dot_general(
        wt, x, (((1,), (1,)), ((), ())),
        preferred_element_type=jnp.float32)
    logits_ref[...] = logits_t

    sub = jax.lax.broadcasted_iota(jnp.int32, logits_t.shape, 0)
    m1 = jnp.max(logits_t, axis=0, keepdims=True)
    i1 = jnp.min(jnp.where(logits_t == m1, sub, NUM_EXPERTS), axis=0,
                 keepdims=True)
    masked = jnp.where(sub == i1, -jnp.inf, logits_t)
    m2 = jnp.max(masked, axis=0, keepdims=True)
    i2 = jnp.min(jnp.where(masked == m2, sub, NUM_EXPERTS), axis=0,
                 keepdims=True)

    e2 = jnp.exp(m2 - m1)
    denom = 1.0 + e2
    rw_ref[...] = jnp.concatenate([1.0 / denom, e2 / denom], axis=0)
    idx_ref[...] = jnp.concatenate([i1, i2], axis=0)


def kernel(hidden_states, W_gate):
    B, S, H = hidden_states.shape
    T = B * S
    x = hidden_states.reshape(T, H)
    wt = W_gate.T
    grid = (T // BLOCK,)

    logits_t, rw_t, idx_t = pl.pallas_call(
        _router_block,
        grid=grid,
        in_specs=[
            pl.BlockSpec((BLOCK, H), lambda i: (i, 0)),
            pl.BlockSpec((NUM_EXPERTS, H), lambda i: (0, 0)),
        ],
        out_specs=[
            pl.BlockSpec((NUM_EXPERTS, BLOCK), lambda i: (0, i)),
            pl.BlockSpec((TOP_K, BLOCK), lambda i: (0, i)),
            pl.BlockSpec((TOP_K, BLOCK), lambda i: (0, i)),
        ],
        out_shape=[
            jax.ShapeDtypeStruct((NUM_EXPERTS, T), jnp.float32),
            jax.ShapeDtypeStruct((TOP_K, T), jnp.float32),
            jax.ShapeDtypeStruct((TOP_K, T), jnp.int32),
        ],
        compiler_params=pltpu.CompilerParams(
            dimension_semantics=("arbitrary",),
        ),
    )(x, wt)

    return (rw_t.T.reshape(B, S, TOP_K),
            idx_t.T.reshape(B, S, TOP_K),
            logits_t.T.reshape(B, S, NUM_EXPERTS))
